# baseline (speedup 1.0000x reference)
import jax
import jax.numpy as jnp
from jax.experimental import pallas as pl
from jax.experimental.pallas import tpu as pltpu

_N = 16384
_D = 1024
_CHS = (1024, 2048, 4096, 4096, 4096, 1024)
_OFFS = (0, 1024, 3072, 7168, 11264, 15360)
_NCHUNK = len(_CHS)
_BUF_IDX = (0, 1, 2, 3, 4, 0)


def _body(x_hbm, o_hbm, *rest):
    bufs = rest[:5]
    insem, outsem = rest[5], rest[6]

    def in_copy(k):
        return pltpu.make_async_copy(
            x_hbm.at[pl.ds(_OFFS[k], _CHS[k])], bufs[_BUF_IDX[k]], insem.at[k])

    def out_copy(k):
        return pltpu.make_async_copy(
            bufs[_BUF_IDX[k]], o_hbm.at[pl.ds(_OFFS[k], _CHS[k])], outsem.at[k])

    for k in range(5):
        in_copy(k).start()
    for k in range(_NCHUNK):
        in_copy(k).wait()
        buf = bufs[_BUF_IDX[k]]
        row_col = (jax.lax.broadcasted_iota(jnp.int32, (_CHS[k], 1), 0)
                   + _OFFS[k]).astype(jnp.float32)
        buf[...] = buf[...] * 2.0 + row_col
        out_copy(k).start()
        if k == 2:
            out_copy(0).wait()
            in_copy(5).start()
    for k in range(1, _NCHUNK):
        out_copy(k).wait()


def kernel(input_tensor):
    return pl.pallas_call(
        _body,
        in_specs=[pl.BlockSpec(memory_space=pl.ANY)],
        out_specs=pl.BlockSpec(memory_space=pl.ANY),
        out_shape=jax.ShapeDtypeStruct((_N, _D), input_tensor.dtype),
        scratch_shapes=(
            [pltpu.VMEM((_CHS[k], _D), jnp.float32) for k in range(5)]
            + [pltpu.SemaphoreType.DMA((_NCHUNK,)),
               pltpu.SemaphoreType.DMA((_NCHUNK,))]
        ),
        compiler_params=pltpu.CompilerParams(
            vmem_limit_bytes=64 * 1024 * 1024,
        ),
    )(input_tensor)
